# baseline (speedup 1.0000x reference)
import functools

import jax
import jax.numpy as jnp
from jax import lax
from jax.experimental import pallas as pl
from jax.experimental.pallas import tpu as pltpu
from jax.experimental.pallas import tpu_sc as plsc

B = 4096
L = 200
V = 1000
D = 64

NC, NS, LANES = 2, 16, 16
NW = NC * NS
TW = B // NW * L
RW = B // NW
DT = 8
CL = 2
NCH = L // CL
NBUF = 2
VP = V + 8
ZROW = V


def _sc_body(table_hbm, tok_hbm, ts_hbm, lut_hbm, out_hbm,
             table_v, lut_v, ts_v, tok_v, stage_v, p0, p1):
    psems = (p0, p1)
    wid = lax.axis_index("s") * NC + lax.axis_index("c")
    tok_base = wid * TW
    row_base = wid * RW

    pltpu.sync_copy(table_hbm, table_v)
    pltpu.sync_copy(lut_hbm, lut_v)
    pltpu.sync_copy(ts_hbm.at[pl.ds(row_base, RW)], ts_v)
    pltpu.sync_copy(tok_hbm.at[pl.ds(tok_base, TW)], tok_v)

    iota = lax.iota(jnp.int32, LANES)
    iotaL = iota * L

    def step(och, carry):
        for b in range(NBUF):
            ch = och * NBUF + b
            svec = stage_v.at[b]
            @pl.when(och > 0)
            def _drain():
                pltpu.make_async_copy(
                    svec, out_hbm.at[pl.ds(0, CL), :, pl.ds(0, 1)],
                    psems[b]).wait()
            for cl in range(CL):
                l = ch * CL + cl

                def inner(bv, car):
                    tok = plsc.load_gather(
                        tok_v, [bv * (LANES * L) + iotaL + l])
                    ids = plsc.load_gather(lut_v, [tok])
                    tsr = ts_v[pl.ds(bv * LANES, LANES)]
                    fid = jnp.where(l < tsr, ids, ZROW) * D
                    for d in range(D):
                        val = plsc.load_gather(table_v, [fid + d])
                        svec[cl, d // 8, 0, d % 8,
                             pl.ds(bv * LANES, LANES)] = val
                    return car

                lax.fori_loop(0, RW // LANES, inner, 0)
            pltpu.async_copy(
                svec, out_hbm.at[pl.ds(ch * CL, CL), :, pl.ds(wid, 1)],
                psems[b])
        return carry

    lax.fori_loop(0, NCH // NBUF, step, 0)
    for b in range(NBUF):
        pltpu.make_async_copy(
            stage_v.at[b], out_hbm.at[pl.ds(0, CL), :, pl.ds(0, 1)],
            psems[b]).wait()


_sc_gather = functools.partial(
    pl.kernel,
    out_type=jax.ShapeDtypeStruct((L, DT, NW, 8, RW), jnp.float32),
    mesh=plsc.VectorSubcoreMesh(
        core_axis_name="c", subcore_axis_name="s",
        num_cores=NC, num_subcores=NS),
    scratch_types=[
        pltpu.VMEM((VP * D,), jnp.float32),
        pltpu.VMEM((V,), jnp.int32),
        pltpu.VMEM((RW,), jnp.int32),
        pltpu.VMEM((TW,), jnp.int32),
        pltpu.VMEM((NBUF, CL, DT, 1, 8, RW), jnp.float32),
    ] + [pltpu.SemaphoreType.DMA] * NBUF,
    compiler_params=pltpu.CompilerParams(
        needs_layout_passes=False, use_tc_tiling_on_sc=False),
)(_sc_body)


def _tc_body(rl_ref, msl_ref, ts_ref, mask_ref):
    ts = jnp.minimum(jnp.minimum(rl_ref[...], msl_ref[...]), L).astype(jnp.int32)
    ts_ref[...] = ts
    col = lax.broadcasted_iota(jnp.int32, (B, L), 1)
    mask_ref[...] = col < ts


_tc_mask = pl.pallas_call(
    _tc_body,
    out_shape=(
        jax.ShapeDtypeStruct((B, 1), jnp.int32),
        jax.ShapeDtypeStruct((B, L), jnp.bool_),
    ),
)


def kernel(tokens, row_lengths, max_sequence_length, lookup_table, embeddings):
    msl = jnp.asarray(max_sequence_length, jnp.int32).reshape(1, 1)
    ts2d, mask = _tc_mask(row_lengths.reshape(B, 1).astype(jnp.int32), msl)
    time_steps = ts2d.reshape(B)
    table_ext = jnp.concatenate(
        [embeddings, jnp.zeros((VP - V, D), jnp.float32)], axis=0)
    out_pht = _sc_gather(table_ext.reshape(VP * D), tokens.reshape(B * L),
                         time_steps, lookup_table)
    out = out_pht.transpose(2, 4, 0, 1, 3)
    return out.reshape(B, L, D), mask, time_steps
